# baseline (speedup 1.0000x reference)
import functools

import jax
import jax.numpy as jnp
from jax import lax
from jax.experimental import pallas as pl
from jax.experimental.pallas import tpu as pltpu
from jax.experimental.pallas import tpu_sc as plsc

_B = 16384
_D = 64
_NC = 2
_NS = 16
_NW = _NC * _NS
_BPW = _B // _NW
_CHUNK = 128
_NCH = _BPW // _CHUNK
_LANES = 16


def _transe_body(ent_hbm, rel_hbm, h_hbm, r_hbm, t_hbm, out_hbm,
                 hidx, ridx, tidx, eh, er, et, sem):
    wid = lax.axis_index("s") * _NC + lax.axis_index("c")
    base = wid * _BPW

    for j in range(_NCH):
        off = base + j * _CHUNK
        pltpu.sync_copy(h_hbm.at[pl.ds(off, _CHUNK)], hidx.at[j])
        pltpu.sync_copy(r_hbm.at[pl.ds(off, _CHUNK)], ridx.at[j])
        pltpu.sync_copy(t_hbm.at[pl.ds(off, _CHUNK)], tidx.at[j])

    copies = []
    for j in range(_NCH):
        dst = pl.ds(j * _CHUNK, _CHUNK)
        copies.append(pltpu.async_copy(ent_hbm.at[hidx.at[j]], eh.at[dst], sem))
        copies.append(pltpu.async_copy(ent_hbm.at[tidx.at[j]], et.at[dst], sem))
        copies.append(pltpu.async_copy(rel_hbm.at[ridx.at[j]], er.at[dst], sem))
    for cp in copies:
        cp.wait()

    def row_step(i, carry):
        for c in range(_D // _LANES):
            s = pl.ds(c * _LANES, _LANES)
            eh[i, s] = eh[i, s] + er[i, s] - et[i, s]
        return carry
    lax.fori_loop(0, _BPW, row_step, 0)

    pltpu.sync_copy(eh, out_hbm.at[pl.ds(base, _BPW)])


@functools.partial(
    pl.kernel,
    out_type=jax.ShapeDtypeStruct((_B, _D), jnp.float32),
    mesh=plsc.VectorSubcoreMesh(core_axis_name="c", subcore_axis_name="s"),
    compiler_params=pltpu.CompilerParams(use_tc_tiling_on_sc=False),
    scratch_types=[
        pltpu.VMEM((_NCH, _CHUNK), jnp.int32),
        pltpu.VMEM((_NCH, _CHUNK), jnp.int32),
        pltpu.VMEM((_NCH, _CHUNK), jnp.int32),
        pltpu.VMEM((_BPW, _D), jnp.float32),
        pltpu.VMEM((_BPW, _D), jnp.float32),
        pltpu.VMEM((_BPW, _D), jnp.float32),
        pltpu.SemaphoreType.DMA,
    ],
)
def _transe(ent_hbm, rel_hbm, h_hbm, r_hbm, t_hbm, out_hbm,
            hidx, ridx, tidx, eh, er, et, sem):
    _transe_body(ent_hbm, rel_hbm, h_hbm, r_hbm, t_hbm, out_hbm,
                 hidx, ridx, tidx, eh, er, et, sem)


def kernel(h, r, t, entity_embeddings, relation_embeddings):
    return _transe(entity_embeddings, relation_embeddings, h, r, t)


# speedup vs baseline: 1.6069x; 1.6069x over previous
import functools

import jax
import jax.numpy as jnp
from jax import lax
from jax.experimental import pallas as pl
from jax.experimental.pallas import tpu as pltpu
from jax.experimental.pallas import tpu_sc as plsc

_B = 16384
_D = 64
_NC = 2
_NS = 16
_NW = _NC * _NS
_BPW = _B // _NW
_CH = 32
_NCHK = _BPW // _CH
_LANES = 16


def _transe_body(ent_hbm, rel_hbm, h_hbm, r_hbm, t_hbm, out_hbm,
                 hv, rv, tv, gh, gr, gt, ob, sem):
    wid = lax.axis_index("s") * _NC + lax.axis_index("c")
    base = wid * _BPW

    pltpu.sync_copy(h_hbm.at[pl.ds(base, _BPW)], hv)
    pltpu.sync_copy(r_hbm.at[pl.ds(base, _BPW)], rv)
    pltpu.sync_copy(t_hbm.at[pl.ds(base, _BPW)], tv)

    lanes = lax.iota(jnp.int32, _LANES)
    zero = jnp.zeros((_LANES,), jnp.int32)

    def chunk_step(j, carry):
        co = j * _CH
        cps = []
        for g in range(_CH // _LANES):
            s = pl.ds(co + g * _LANES, _LANES)
            hvv = hv[s]
            rvv = rv[s]
            tvv = tv[s]
            for l in range(_LANES):
                m = lanes == l
                he = jnp.max(jnp.where(m, hvv, zero))
                re_ = jnp.max(jnp.where(m, rvv, zero))
                te = jnp.max(jnp.where(m, tvv, zero))
                e = g * _LANES + l
                cps.append(pltpu.async_copy(ent_hbm.at[pl.ds(he, 1)],
                                            gh.at[pl.ds(e, 1)], sem))
                cps.append(pltpu.async_copy(rel_hbm.at[pl.ds(re_, 1)],
                                            gr.at[pl.ds(e, 1)], sem))
                cps.append(pltpu.async_copy(ent_hbm.at[pl.ds(te, 1)],
                                            gt.at[pl.ds(e, 1)], sem))
        for cp in cps:
            cp.wait()
        for e in range(_CH):
            for c in range(_D // _LANES):
                cs = pl.ds(c * _LANES, _LANES)
                ob[e, cs] = gh[e, cs] + gr[e, cs] - gt[e, cs]
        pltpu.sync_copy(ob, out_hbm.at[pl.ds(base + co, _CH)])
        return carry
    lax.fori_loop(0, _NCHK, chunk_step, 0)


@functools.partial(
    pl.kernel,
    out_type=jax.ShapeDtypeStruct((_B, _D), jnp.float32),
    mesh=plsc.VectorSubcoreMesh(core_axis_name="c", subcore_axis_name="s"),
    compiler_params=pltpu.CompilerParams(needs_layout_passes=False),
    scratch_types=[
        pltpu.VMEM((_BPW,), jnp.int32),
        pltpu.VMEM((_BPW,), jnp.int32),
        pltpu.VMEM((_BPW,), jnp.int32),
        pltpu.VMEM((_CH, _D), jnp.float32),
        pltpu.VMEM((_CH, _D), jnp.float32),
        pltpu.VMEM((_CH, _D), jnp.float32),
        pltpu.VMEM((_CH, _D), jnp.float32),
        pltpu.SemaphoreType.DMA,
    ],
)
def _transe(ent_hbm, rel_hbm, h_hbm, r_hbm, t_hbm, out_hbm,
            hv, rv, tv, gh, gr, gt, ob, sem):
    _transe_body(ent_hbm, rel_hbm, h_hbm, r_hbm, t_hbm, out_hbm,
                 hv, rv, tv, gh, gr, gt, ob, sem)


def kernel(h, r, t, entity_embeddings, relation_embeddings):
    return _transe(entity_embeddings, relation_embeddings, h, r, t)
